# Initial kernel scaffold; baseline (speedup 1.0000x reference)
import jax
import jax.numpy as jnp
from jax.experimental import pallas as pl


def kernel(X, edge_index, W1, b1, W_mu, b_mu, W_ls, b_ls):
    raise NotImplementedError("write your pallas kernel here")


# speedup vs baseline: 24.5054x; 24.5054x over previous
import functools

import jax
import jax.numpy as jnp
from jax import lax
from jax.experimental import pallas as pl
from jax.experimental.pallas import tpu as pltpu
from jax.experimental.pallas import tpu_sc as plsc

N = 10000
E = 320000
D_IN, DW = 128, 128
D_H = 32

NC, NS = 2, 16
CHUNK = 128
CPT = 80
EPAD = NC * NS * CPT * CHUNK
PAD_ROWS = 240
NP = N + PAD_ROWS
RPT = NP // NS
ZR = 160

_f32 = jnp.float32
_mesh = plsc.VectorSubcoreMesh(
    core_axis_name="c", subcore_axis_name="s", num_cores=NC, num_subcores=NS)


def _fill16(ref, n, val):
  def body(i, _):
    ref[pl.ds(i * 16, 16)] = jnp.full((16,), val, _f32)
    return None
  lax.fori_loop(0, n, body, None)


@functools.partial(
    pl.kernel,
    out_type=jax.ShapeDtypeStruct((NC, NP), _f32),
    mesh=_mesh,
    scratch_types=[
        pltpu.VMEM((CPT, CHUNK), jnp.int32),
        pltpu.VMEM((CHUNK,), _f32),
        pltpu.VMEM((RPT,), _f32),
        pltpu.VMEM_SHARED((NP,), _f32),
    ],
)
def _deg_kernel(dst_hbm, out_hbm, didx, ones, zrow, deg_sh):
  cid = lax.axis_index("c")
  sid = lax.axis_index("s")
  _fill16(ones, CHUNK // 16, 1.0)
  _fill16(zrow, RPT // 16, 0.0)
  pltpu.sync_copy(zrow, deg_sh.at[pl.ds(sid * RPT, RPT)])
  plsc.subcore_barrier()
  pltpu.sync_copy(dst_hbm.at[cid, sid], didx)

  def body(j, _):
    pltpu.sync_copy(ones, deg_sh.at[didx.at[j]], add=True)
    return None
  lax.fori_loop(0, CPT, body, None)
  plsc.subcore_barrier()
  pltpu.sync_copy(deg_sh.at[pl.ds(sid * RPT, RPT)],
                  out_hbm.at[cid, pl.ds(sid * RPT, RPT)])


@functools.partial(
    pl.kernel,
    out_type=jax.ShapeDtypeStruct((NC, NP * D_H), _f32),
    mesh=_mesh,
    scratch_types=[
        pltpu.VMEM((CPT, CHUNK), jnp.int32),
        pltpu.VMEM((CPT, CHUNK), jnp.int32),
        pltpu.VMEM((CHUNK, DW), _f32),
        pltpu.VMEM((CHUNK * D_H,), _f32),
        pltpu.VMEM_SHARED((NP, DW), _f32),
    ],
)
def _agg_kernel(h_hbm, src_hbm, dst_hbm, out_hbm,
                sidx, didx, rows0, pk, acc_sh):
  cid = lax.axis_index("c")
  sid = lax.axis_index("s")

  def zbody(i, _):
    r = i // 8
    c = (i % 8) * 16
    rows0[r, pl.ds(c, 16)] = jnp.zeros((16,), _f32)
    return None
  lax.fori_loop(0, CHUNK * 8, zbody, None)
  for q in range(RPT // CHUNK):
    pltpu.sync_copy(rows0, acc_sh.at[pl.ds(sid * RPT + q * CHUNK, CHUNK)])
  pltpu.sync_copy(src_hbm.at[cid, sid], sidx)
  pltpu.sync_copy(dst_hbm.at[cid, sid], didx)
  plsc.subcore_barrier()

  def body(j, _):
    pltpu.sync_copy(h_hbm.at[sidx.at[j]], rows0)
    pltpu.sync_copy(rows0, acc_sh.at[didx.at[j]], add=True)
    return None
  lax.fori_loop(0, CPT, body, None)

  plsc.subcore_barrier()
  for q in range(RPT // CHUNK):
    base = sid * RPT + q * CHUNK
    pltpu.sync_copy(acc_sh.at[pl.ds(base, CHUNK)], rows0)

    def pbody(r, _):
      pk[pl.ds(r * D_H, 16)] = rows0[r, pl.ds(0, 16)]
      pk[pl.ds(r * D_H + 16, 16)] = rows0[r, pl.ds(16, 16)]
      return None
    lax.fori_loop(0, CHUNK, pbody, None)
    pltpu.sync_copy(pk, out_hbm.at[cid, pl.ds(base * D_H, CHUNK * D_H)])


def _mm_scale_body(deg_ref, x_ref, w_ref, dis_ref, hs_ref):
  deg = deg_ref[0] + deg_ref[1]
  dis = lax.rsqrt(deg[:N] + 1.0)
  h = jnp.dot(x_ref[...], w_ref[...], preferred_element_type=_f32)
  dis_ref[...] = dis
  hs_ref[...] = h * dis


def _relu_scale_body(aggp_ref, hs_ref, dis_ref, b1_ref, h1s_ref):
  a = aggp_ref[0][:N] + aggp_ref[1][:N]
  a128 = jnp.concatenate([a, jnp.zeros((N, DW - D_H), _f32)], axis=1)
  dis = dis_ref[...]
  h1 = jnp.maximum(dis * (a128 + hs_ref[...]) + b1_ref[...], 0.0)
  h1s_ref[...] = h1 * dis


def _final_body(aggp_ref, h1s_ref, dis_ref, w_ref, b_ref, out_ref):
  a = aggp_ref[0][:N] + aggp_ref[1][:N]
  a2 = dis_ref[...] * (a + h1s_ref[:, :D_H])
  out_ref[...] = jnp.dot(a2, w_ref[...], preferred_element_type=_f32) + b_ref[...]


def kernel(X, edge_index, W1, b1, W_mu, b_mu, W_ls, b_ls):
  src = edge_index[0]
  dst = edge_index[1]
  pad = EPAD - E
  ar = jnp.arange(pad, dtype=edge_index.dtype)
  src_p = jnp.concatenate([src, ar % N]).reshape(NC, NS, CPT, CHUNK)
  dst_p = jnp.concatenate([dst, N + ar % PAD_ROWS]).reshape(NC, NS, CPT, CHUNK)

  deg_p = _deg_kernel(dst_p)
  deg3 = deg_p.reshape(NC, NP, 1)

  d_h = W1.shape[1]
  w1p = jnp.pad(W1, ((0, 0), (0, DW - d_h)))
  dis, hs = pl.pallas_call(
      _mm_scale_body,
      out_shape=(jax.ShapeDtypeStruct((N, 1), _f32),
                 jax.ShapeDtypeStruct((N, DW), _f32)),
  )(deg3, X, w1p)

  agg1 = _agg_kernel(hs, src_p, dst_p).reshape(NC, NP, D_H)

  b1p = jnp.pad(b1.reshape(1, d_h), ((0, 0), (0, DW - d_h)))
  h1s = pl.pallas_call(
      _relu_scale_body,
      out_shape=jax.ShapeDtypeStruct((N, DW), _f32),
  )(agg1, hs, dis, b1p)

  agg2 = _agg_kernel(h1s, src_p, dst_p).reshape(NC, NP, D_H)

  k = b_mu.shape[0]
  w_cat = jnp.concatenate([W_mu, W_ls], axis=1)
  b_cat = jnp.concatenate([b_mu, b_ls]).reshape(1, 2 * k)
  out = pl.pallas_call(
      _final_body,
      out_shape=jax.ShapeDtypeStruct((N, 2 * k), _f32),
  )(agg2, h1s, dis, w_cat, b_cat)

  return (out[:, :k], out[:, k:])


# speedup vs baseline: 33.9993x; 1.3874x over previous
import functools

import jax
import jax.numpy as jnp
from jax import lax
from jax.experimental import pallas as pl
from jax.experimental.pallas import tpu as pltpu
from jax.experimental.pallas import tpu_sc as plsc

N = 10000
E = 320000
D_IN, DW = 128, 128
D_H = 32

NC, NS = 2, 16
CHUNK = 128
CPT = 80
PH, CPP = 2, 40
EPAD = NC * NS * CPT * CHUNK
PAD_ROWS = 240
NP = N + PAD_ROWS
RPT = NP // NS
ZR = 160

_f32 = jnp.float32
_mesh = plsc.VectorSubcoreMesh(
    core_axis_name="c", subcore_axis_name="s", num_cores=NC, num_subcores=NS)


def _fill16(ref, n, val):
  def body(i, _):
    ref[pl.ds(i * 16, 16)] = jnp.full((16,), val, _f32)
    return None
  lax.fori_loop(0, n, body, None)


@functools.partial(
    pl.kernel,
    out_type=jax.ShapeDtypeStruct((NC, NP), _f32),
    mesh=_mesh,
    scratch_types=[
        pltpu.VMEM((CPT, CHUNK), jnp.int32),
        pltpu.VMEM((CHUNK,), _f32),
        pltpu.VMEM((RPT,), _f32),
        pltpu.VMEM_SHARED((NP,), _f32),
    ],
)
def _deg_kernel(dst_hbm, out_hbm, didx, ones, zrow, deg_sh):
  cid = lax.axis_index("c")
  sid = lax.axis_index("s")
  _fill16(ones, CHUNK // 16, 1.0)
  _fill16(zrow, RPT // 16, 0.0)
  pltpu.sync_copy(zrow, deg_sh.at[pl.ds(sid * RPT, RPT)])
  plsc.subcore_barrier()
  pltpu.sync_copy(dst_hbm.at[cid, sid], didx)

  def body(j, _):
    pltpu.sync_copy(ones, deg_sh.at[didx.at[j]], add=True)
    return None
  lax.fori_loop(0, CPT, body, None)
  plsc.subcore_barrier()
  pltpu.sync_copy(deg_sh.at[pl.ds(sid * RPT, RPT)],
                  out_hbm.at[cid, pl.ds(sid * RPT, RPT)])


@functools.partial(
    pl.kernel,
    out_type=jax.ShapeDtypeStruct((NC, NP * D_H), _f32),
    mesh=_mesh,
    scratch_types=[
        pltpu.VMEM((CPP, CHUNK), jnp.int32),
        pltpu.VMEM((CPP, CHUNK), jnp.int32),
        pltpu.VMEM((CHUNK, DW), _f32),
        pltpu.VMEM((CHUNK, DW), _f32),
        pltpu.VMEM((CHUNK * D_H,), _f32),
        pltpu.VMEM_SHARED((NP, DW), _f32),
        pltpu.SemaphoreType.DMA,
        pltpu.SemaphoreType.DMA,
    ],
)
def _agg_kernel(h_hbm, src_hbm, dst_hbm, out_hbm,
                sidx, didx, rows0, rows1, pk, acc_sh, gsem0, gsem1):
  cid = lax.axis_index("c")
  sid = lax.axis_index("s")

  def zbody(i, _):
    r = i // 8
    c = (i % 8) * 16
    rows0[r, pl.ds(c, 16)] = jnp.zeros((16,), _f32)
    return None
  lax.fori_loop(0, CHUNK * 8, zbody, None)
  for q in range(RPT // CHUNK):
    pltpu.sync_copy(rows0, acc_sh.at[pl.ds(sid * RPT + q * CHUNK, CHUNK)])
  plsc.subcore_barrier()

  for ph in range(PH):
    pltpu.sync_copy(src_hbm.at[cid, sid, pl.ds(ph * CPP, CPP)], sidx)
    pltpu.sync_copy(dst_hbm.at[cid, sid, pl.ds(ph * CPP, CPP)], didx)
    pltpu.async_copy(h_hbm.at[sidx.at[0]], rows0, gsem0)
    pltpu.async_copy(h_hbm.at[sidx.at[1]], rows1, gsem1)

    def body(g, _):
      for b, rows, sem in ((0, rows0, gsem0), (1, rows1, gsem1)):
        j = 2 * g + b
        pltpu.make_async_copy(h_hbm.at[sidx.at[j]], rows, sem).wait()
        pltpu.sync_copy(rows, acc_sh.at[didx.at[j]], add=True)

        @pl.when(j + 2 < CPP)
        def _():
          pltpu.async_copy(h_hbm.at[sidx.at[j + 2]], rows, sem)
      return None
    lax.fori_loop(0, CPP // 2, body, None)

  plsc.subcore_barrier()
  for q in range(RPT // CHUNK):
    base = sid * RPT + q * CHUNK
    pltpu.sync_copy(acc_sh.at[pl.ds(base, CHUNK)], rows0)

    def pbody(r, _):
      pk[pl.ds(r * D_H, 16)] = rows0[r, pl.ds(0, 16)]
      pk[pl.ds(r * D_H + 16, 16)] = rows0[r, pl.ds(16, 16)]
      return None
    lax.fori_loop(0, CHUNK, pbody, None)
    pltpu.sync_copy(pk, out_hbm.at[cid, pl.ds(base * D_H, CHUNK * D_H)])


def _mm_scale_body(deg_ref, x_ref, w_ref, dis_ref, hs_ref):
  deg = deg_ref[0] + deg_ref[1]
  dis = lax.rsqrt(deg[:N] + 1.0)
  h = jnp.dot(x_ref[...], w_ref[...], preferred_element_type=_f32)
  dis_ref[...] = dis
  hs_ref[...] = h * dis


def _relu_scale_body(aggp_ref, hs_ref, dis_ref, b1_ref, h1s_ref):
  a = aggp_ref[0][:N] + aggp_ref[1][:N]
  a128 = jnp.concatenate([a, jnp.zeros((N, DW - D_H), _f32)], axis=1)
  dis = dis_ref[...]
  h1 = jnp.maximum(dis * (a128 + hs_ref[...]) + b1_ref[...], 0.0)
  h1s_ref[...] = h1 * dis


def _final_body(aggp_ref, h1s_ref, dis_ref, w_ref, b_ref, out_ref):
  a = aggp_ref[0][:N] + aggp_ref[1][:N]
  a2 = dis_ref[...] * (a + h1s_ref[:, :D_H])
  out_ref[...] = jnp.dot(a2, w_ref[...], preferred_element_type=_f32) + b_ref[...]


def kernel(X, edge_index, W1, b1, W_mu, b_mu, W_ls, b_ls):
  src = edge_index[0]
  dst = edge_index[1]
  pad = EPAD - E
  ar = jnp.arange(pad, dtype=edge_index.dtype)
  src_f = jnp.concatenate([src, ar % N])
  dst_f = jnp.concatenate([dst, N + ar % PAD_ROWS])
  src_p = src_f.reshape(NC, NS, CPT, CHUNK)
  dst_p = dst_f.reshape(NC, NS, CPT, CHUNK)

  deg_p = _deg_kernel(dst_p)
  deg3 = deg_p.reshape(NC, NP, 1)

  d_h = W1.shape[1]
  w1p = jnp.pad(W1, ((0, 0), (0, DW - d_h)))
  dis, hs = pl.pallas_call(
      _mm_scale_body,
      out_shape=(jax.ShapeDtypeStruct((N, 1), _f32),
                 jax.ShapeDtypeStruct((N, DW), _f32)),
  )(deg3, X, w1p)

  agg1 = _agg_kernel(hs, src_p, dst_p).reshape(NC, NP, D_H)

  b1p = jnp.pad(b1.reshape(1, d_h), ((0, 0), (0, DW - d_h)))
  h1s = pl.pallas_call(
      _relu_scale_body,
      out_shape=jax.ShapeDtypeStruct((N, DW), _f32),
  )(agg1, hs, dis, b1p)

  agg2 = _agg_kernel(h1s, src_p, dst_p).reshape(NC, NP, D_H)

  k = b_mu.shape[0]
  w_cat = jnp.concatenate([W_mu, W_ls], axis=1)
  b_cat = jnp.concatenate([b_mu, b_ls]).reshape(1, 2 * k)
  out = pl.pallas_call(
      _final_body,
      out_shape=jax.ShapeDtypeStruct((N, 2 * k), _f32),
  )(agg2, h1s, dis, w_cat, b_cat)

  return (out[:, :k], out[:, k:])


# speedup vs baseline: 36.6530x; 1.0781x over previous
import functools

import jax
import jax.numpy as jnp
from jax import lax
from jax.experimental import pallas as pl
from jax.experimental.pallas import tpu as pltpu
from jax.experimental.pallas import tpu_sc as plsc

N = 10000
E = 320000
D_IN, DW = 128, 128
D_H = 32

NC, NS = 2, 16
CHUNK = 128
CPT = 80
PH, CPP = 2, 40
EPAD = NC * NS * CPT * CHUNK
PAD_ROWS = 240
NP = N + PAD_ROWS
RPT = NP // NS
ZR = 160

_f32 = jnp.float32
_mesh = plsc.VectorSubcoreMesh(
    core_axis_name="c", subcore_axis_name="s", num_cores=NC, num_subcores=NS)


def _fill16(ref, n, val):
  def body(i, _):
    ref[pl.ds(i * 16, 16)] = jnp.full((16,), val, _f32)
    return None
  lax.fori_loop(0, n, body, None)


@functools.partial(
    pl.kernel,
    out_type=jax.ShapeDtypeStruct((NC, NP), _f32),
    mesh=_mesh,
    scratch_types=[
        pltpu.VMEM((CPT, CHUNK), jnp.int32),
        pltpu.VMEM((CHUNK,), _f32),
        pltpu.VMEM((RPT,), _f32),
        pltpu.VMEM_SHARED((NP,), _f32),
    ],
)
def _deg_kernel(dst_hbm, out_hbm, didx, ones, zrow, deg_sh):
  cid = lax.axis_index("c")
  sid = lax.axis_index("s")
  _fill16(ones, CHUNK // 16, 1.0)
  _fill16(zrow, RPT // 16, 0.0)
  pltpu.sync_copy(zrow, deg_sh.at[pl.ds(sid * RPT, RPT)])
  plsc.subcore_barrier()
  pltpu.sync_copy(dst_hbm.at[cid, sid], didx)

  def body(j, _):
    pltpu.sync_copy(ones, deg_sh.at[didx.at[j]], add=True)
    return None
  lax.fori_loop(0, CPT, body, None)
  plsc.subcore_barrier()
  pltpu.sync_copy(deg_sh.at[pl.ds(sid * RPT, RPT)],
                  out_hbm.at[cid, pl.ds(sid * RPT, RPT)])


@functools.partial(
    pl.kernel,
    out_type=jax.ShapeDtypeStruct((NC, NP, DW), _f32),
    mesh=_mesh,
    scratch_types=[
        pltpu.VMEM((CPP, CHUNK), jnp.int32),
        pltpu.VMEM((CPP, CHUNK), jnp.int32),
        pltpu.VMEM((CHUNK, DW), _f32),
        pltpu.VMEM((CHUNK, DW), _f32),
        pltpu.VMEM_SHARED((NP, DW), _f32),
        pltpu.SemaphoreType.DMA,
        pltpu.SemaphoreType.DMA,
    ],
)
def _agg_kernel(h_hbm, src_hbm, dst_hbm, out_hbm,
                sidx, didx, rows0, rows1, acc_sh, gsem0, gsem1):
  cid = lax.axis_index("c")
  sid = lax.axis_index("s")

  def zbody(i, _):
    r = i // 8
    c = (i % 8) * 16
    rows0[r, pl.ds(c, 16)] = jnp.zeros((16,), _f32)
    return None
  lax.fori_loop(0, CHUNK * 8, zbody, None)
  for q in range(RPT // CHUNK):
    pltpu.async_copy(rows0, acc_sh.at[pl.ds(sid * RPT + q * CHUNK, CHUNK)],
                     gsem0)
  for q in range(RPT // CHUNK):
    pltpu.make_async_copy(
        rows0, acc_sh.at[pl.ds(sid * RPT + q * CHUNK, CHUNK)], gsem0).wait()
  plsc.subcore_barrier()

  for ph in range(PH):
    pltpu.sync_copy(src_hbm.at[cid, sid, pl.ds(ph * CPP, CPP)], sidx)
    pltpu.sync_copy(dst_hbm.at[cid, sid, pl.ds(ph * CPP, CPP)], didx)
    pltpu.async_copy(h_hbm.at[sidx.at[0]], rows0, gsem0)
    pltpu.async_copy(h_hbm.at[sidx.at[1]], rows1, gsem1)

    def body(g, _):
      for b, rows, sem in ((0, rows0, gsem0), (1, rows1, gsem1)):
        j = 2 * g + b
        pltpu.make_async_copy(h_hbm.at[sidx.at[j]], rows, sem).wait()
        pltpu.sync_copy(rows, acc_sh.at[didx.at[j]], add=True)

        @pl.when(j + 2 < CPP)
        def _():
          pltpu.async_copy(h_hbm.at[sidx.at[j + 2]], rows, sem)
      return None
    lax.fori_loop(0, CPP // 2, body, None)

  plsc.subcore_barrier()
  pltpu.sync_copy(acc_sh.at[pl.ds(sid * RPT, RPT)],
                  out_hbm.at[cid, pl.ds(sid * RPT, RPT)])


def _mm_scale_body(deg_ref, x_ref, w_ref, dis_ref, hs_ref):
  deg = deg_ref[0] + deg_ref[1]
  dis = lax.rsqrt(deg[:N] + 1.0)
  h = jnp.dot(x_ref[...], w_ref[...], preferred_element_type=_f32)
  dis_ref[...] = dis
  hs_ref[...] = jnp.concatenate(
      [h * dis, jnp.zeros((N, DW - D_H), _f32)], axis=1)


def _relu_scale_body(aggp_ref, hs_ref, dis_ref, b1_ref, h1s_ref):
  a = aggp_ref[0][:N] + aggp_ref[1][:N]
  b1p = jnp.concatenate(
      [b1_ref[...], jnp.zeros((1, DW - D_H), _f32)], axis=1)
  dis = dis_ref[...]
  h1 = jnp.maximum(dis * (a + hs_ref[...]) + b1p, 0.0)
  h1s_ref[...] = h1 * dis


def _final_body(aggp_ref, h1s_ref, dis_ref, w_ref, b_ref, out_ref):
  a = aggp_ref[0][:N, :D_H] + aggp_ref[1][:N, :D_H]
  a2 = dis_ref[...] * (a + h1s_ref[:, :D_H])
  out_ref[...] = jnp.dot(a2, w_ref[...], preferred_element_type=_f32) + b_ref[...]


def kernel(X, edge_index, W1, b1, W_mu, b_mu, W_ls, b_ls):
  src = edge_index[0]
  dst = edge_index[1]
  pad = EPAD - E
  ar = jnp.arange(pad, dtype=edge_index.dtype)
  src_f = jnp.concatenate([src, ar % N])
  dst_f = jnp.concatenate([dst, N + ar % PAD_ROWS])
  src_p = src_f.reshape(NC, NS, CPT, CHUNK)
  dst_p = dst_f.reshape(NC, NS, CPT, CHUNK)

  deg_p = _deg_kernel(dst_p)
  deg3 = deg_p.reshape(NC, NP, 1)

  dis, hs = pl.pallas_call(
      _mm_scale_body,
      out_shape=(jax.ShapeDtypeStruct((N, 1), _f32),
                 jax.ShapeDtypeStruct((N, DW), _f32)),
  )(deg3, X, W1)

  agg1 = _agg_kernel(hs, src_p, dst_p)

  h1s = pl.pallas_call(
      _relu_scale_body,
      out_shape=jax.ShapeDtypeStruct((N, DW), _f32),
  )(agg1, hs, dis, b1.reshape(1, D_H))

  agg2 = _agg_kernel(h1s, src_p, dst_p)

  k = b_mu.shape[0]
  w_cat = jnp.concatenate([W_mu, W_ls], axis=1)
  b_cat = jnp.concatenate([b_mu, b_ls]).reshape(1, 2 * k)
  out = pl.pallas_call(
      _final_body,
      out_shape=jax.ShapeDtypeStruct((N, 2 * k), _f32),
  )(agg2, h1s, dis, w_cat, b_cat)

  return (out[:, :k], out[:, k:])


# speedup vs baseline: 38.4731x; 1.0497x over previous
import functools

import jax
import jax.numpy as jnp
from jax import lax
from jax.experimental import pallas as pl
from jax.experimental.pallas import tpu as pltpu
from jax.experimental.pallas import tpu_sc as plsc

N = 10000
E = 320000
D_IN, DW = 128, 128
D_H = 32

NC, NS = 2, 16
CHUNK = 128
CPT = 80
PH, CPP = 2, 40
EPAD = NC * NS * CPT * CHUNK
PAD_ROWS = 240
NP = N + PAD_ROWS
RPT = NP // NS
ZR = 160

_f32 = jnp.float32
_mesh = plsc.VectorSubcoreMesh(
    core_axis_name="c", subcore_axis_name="s", num_cores=NC, num_subcores=NS)


def _fill16(ref, n, val):
  def body(i, _):
    ref[pl.ds(i * 16, 16)] = jnp.full((16,), val, _f32)
    return None
  lax.fori_loop(0, n, body, None)


@functools.partial(
    pl.kernel,
    out_type=jax.ShapeDtypeStruct((NC, NP), _f32),
    mesh=_mesh,
    scratch_types=[
        pltpu.VMEM((CPT, CHUNK), jnp.int32),
        pltpu.VMEM((CHUNK,), _f32),
        pltpu.VMEM((RPT,), _f32),
        pltpu.VMEM_SHARED((NP,), _f32),
    ],
)
def _deg_kernel(dst_hbm, out_hbm, didx, ones, zrow, deg_sh):
  cid = lax.axis_index("c")
  sid = lax.axis_index("s")
  _fill16(ones, CHUNK // 16, 1.0)
  _fill16(zrow, RPT // 16, 0.0)
  pltpu.sync_copy(zrow, deg_sh.at[pl.ds(sid * RPT, RPT)])
  plsc.subcore_barrier()
  pltpu.sync_copy(dst_hbm.at[cid, sid], didx)

  def body(j, _):
    pltpu.sync_copy(ones, deg_sh.at[didx.at[j]], add=True)
    return None
  lax.fori_loop(0, CPT, body, None)
  plsc.subcore_barrier()
  pltpu.sync_copy(deg_sh.at[pl.ds(sid * RPT, RPT)],
                  out_hbm.at[cid, pl.ds(sid * RPT, RPT)])


@functools.partial(
    pl.kernel,
    out_type=jax.ShapeDtypeStruct((NC, NP, DW), _f32),
    mesh=_mesh,
    scratch_types=[
        pltpu.VMEM((CPP, CHUNK), jnp.int32),
        pltpu.VMEM((CPP, CHUNK), jnp.int32),
        pltpu.VMEM((CHUNK, DW), _f32),
        pltpu.VMEM((CHUNK, DW), _f32),
        pltpu.VMEM_SHARED((NP, DW), _f32),
        pltpu.SemaphoreType.DMA,
        pltpu.SemaphoreType.DMA,
    ],
)
def _agg_kernel(h_hbm, src_hbm, dst_hbm, out_hbm,
                sidx, didx, rows0, rows1, acc_sh, gsem0, gsem1):
  cid = lax.axis_index("c")
  sid = lax.axis_index("s")

  def zbody(i, _):
    r = i // 8
    c = (i % 8) * 16
    rows0[r, pl.ds(c, 16)] = jnp.zeros((16,), _f32)
    return None
  lax.fori_loop(0, CHUNK * 8, zbody, None)
  for q in range(RPT // CHUNK):
    pltpu.async_copy(rows0, acc_sh.at[pl.ds(sid * RPT + q * CHUNK, CHUNK)],
                     gsem0)
  for q in range(RPT // CHUNK):
    pltpu.make_async_copy(
        rows0, acc_sh.at[pl.ds(sid * RPT + q * CHUNK, CHUNK)], gsem0).wait()
  plsc.subcore_barrier()

  for ph in range(PH):
    pltpu.sync_copy(src_hbm.at[cid, sid, pl.ds(ph * CPP, CPP)], sidx)
    pltpu.sync_copy(dst_hbm.at[cid, sid, pl.ds(ph * CPP, CPP)], didx)
    pltpu.async_copy(h_hbm.at[sidx.at[0]], rows0, gsem0)
    pltpu.async_copy(h_hbm.at[sidx.at[1]], rows1, gsem1)

    def body(g, _):
      for b, rows, sem in ((0, rows0, gsem0), (1, rows1, gsem1)):
        j = 2 * g + b
        pltpu.make_async_copy(h_hbm.at[sidx.at[j]], rows, sem).wait()
        pltpu.sync_copy(rows, acc_sh.at[didx.at[j]], add=True)

        @pl.when(j + 2 < CPP)
        def _():
          pltpu.async_copy(h_hbm.at[sidx.at[j + 2]], rows, sem)
      return None
    lax.fori_loop(0, CPP // 2, body, None)

  plsc.subcore_barrier()
  pltpu.sync_copy(acc_sh.at[pl.ds(sid * RPT, RPT)],
                  out_hbm.at[cid, pl.ds(sid * RPT, RPT)])


def _mm_body(x_ref, w_ref, h_ref):
  h = jnp.dot(x_ref[...], w_ref[...], preferred_element_type=_f32)
  h_ref[...] = jnp.concatenate(
      [h, jnp.zeros((N, DW - D_H), _f32)], axis=1)


def _scale_body(deg_ref, h_ref, dis_ref, hs_ref):
  deg = deg_ref[0] + deg_ref[1]
  dis = lax.rsqrt(deg[:N] + 1.0).reshape(N, 1)
  dis_ref[...] = dis
  hs_ref[...] = h_ref[...] * dis


def _relu_scale_body(aggp_ref, hs_ref, dis_ref, b1_ref, h1s_ref):
  a = aggp_ref[0][:N] + aggp_ref[1][:N]
  b1p = jnp.concatenate(
      [b1_ref[...], jnp.zeros((1, DW - D_H), _f32)], axis=1)
  dis = dis_ref[...]
  h1 = jnp.maximum(dis * (a + hs_ref[...]) + b1p, 0.0)
  h1s_ref[...] = h1 * dis


def _final_body(aggp_ref, h1s_ref, dis_ref, w_ref, b_ref, mu_ref, ls_ref):
  a = aggp_ref[0][:N, :D_H] + aggp_ref[1][:N, :D_H]
  a2 = dis_ref[...] * (a + h1s_ref[:, :D_H])
  out = jnp.dot(a2, w_ref[...], preferred_element_type=_f32) + b_ref[...]
  k = out.shape[1] // 2
  mu_ref[...] = out[:, :k]
  ls_ref[...] = out[:, k:]


def kernel(X, edge_index, W1, b1, W_mu, b_mu, W_ls, b_ls):
  src = edge_index[0]
  dst = edge_index[1]
  pad = EPAD - E
  ar = jnp.arange(pad, dtype=edge_index.dtype)
  src_f = jnp.concatenate([src, ar % N])
  dst_f = jnp.concatenate([dst, N + ar % PAD_ROWS])
  src_p = src_f.reshape(NC, NS, CPT, CHUNK)
  dst_p = dst_f.reshape(NC, NS, CPT, CHUNK)

  h128 = pl.pallas_call(
      _mm_body,
      out_shape=jax.ShapeDtypeStruct((N, DW), _f32),
  )(X, W1)
  deg_p = _deg_kernel(dst_p)

  dis, hs = pl.pallas_call(
      _scale_body,
      out_shape=(jax.ShapeDtypeStruct((N, 1), _f32),
                 jax.ShapeDtypeStruct((N, DW), _f32)),
  )(deg_p, h128)

  agg1 = _agg_kernel(hs, src_p, dst_p)

  h1s = pl.pallas_call(
      _relu_scale_body,
      out_shape=jax.ShapeDtypeStruct((N, DW), _f32),
  )(agg1, hs, dis, b1.reshape(1, D_H))

  agg2 = _agg_kernel(h1s, src_p, dst_p)

  k = b_mu.shape[0]
  w_cat = jnp.concatenate([W_mu, W_ls], axis=1)
  b_cat = jnp.concatenate([b_mu, b_ls]).reshape(1, 2 * k)
  mu, ls = pl.pallas_call(
      _final_body,
      out_shape=(jax.ShapeDtypeStruct((N, k), _f32),
                 jax.ShapeDtypeStruct((N, k), _f32)),
  )(agg2, h1s, dis, w_cat, b_cat)

  return (mu, ls)


# speedup vs baseline: 40.8759x; 1.0625x over previous
import functools

import jax
import jax.numpy as jnp
from jax import lax
from jax.experimental import pallas as pl
from jax.experimental.pallas import tpu as pltpu
from jax.experimental.pallas import tpu_sc as plsc

N = 10000
E = 320000
D_IN, DW = 128, 128
D_H = 32

NC, NS = 2, 16
CHUNK = 128
CPT = 80
PH, CPP = 2, 40
EPAD = NC * NS * CPT * CHUNK
PAD_ROWS = 240
NP = N + PAD_ROWS
RPT = NP // NS
ZR = 160

_f32 = jnp.float32
_mesh = plsc.VectorSubcoreMesh(
    core_axis_name="c", subcore_axis_name="s", num_cores=NC, num_subcores=NS)


def _fill16(ref, n, val):
  def body(i, _):
    ref[pl.ds(i * 16, 16)] = jnp.full((16,), val, _f32)
    return None
  lax.fori_loop(0, n, body, None)


@functools.partial(
    pl.kernel,
    out_type=jax.ShapeDtypeStruct((NC, NP), _f32),
    mesh=_mesh,
    scratch_types=[
        pltpu.VMEM((CPT, CHUNK), jnp.int32),
        pltpu.VMEM((CHUNK,), _f32),
        pltpu.VMEM((RPT,), _f32),
        pltpu.VMEM_SHARED((NP,), _f32),
        pltpu.SemaphoreType.DMA,
    ],
)
def _deg_kernel(dst_hbm, out_hbm, didx, ones, zrow, deg_sh, gsem):
  cid = lax.axis_index("c")
  sid = lax.axis_index("s")
  _fill16(ones, CHUNK // 16, 1.0)
  _fill16(zrow, RPT // 16, 0.0)
  pltpu.sync_copy(zrow, deg_sh.at[pl.ds(sid * RPT, RPT)])
  plsc.subcore_barrier()
  pltpu.sync_copy(dst_hbm.at[cid, sid], didx)

  def body(j, _):
    pltpu.async_copy(ones, deg_sh.at[didx.at[j]], gsem, add=True)
    return None
  lax.fori_loop(0, CPT, body, None)

  def drain(j, _):
    pltpu.make_async_copy(ones, deg_sh.at[didx.at[j]], gsem).wait()
    return None
  lax.fori_loop(0, CPT, drain, None)
  plsc.subcore_barrier()
  pltpu.sync_copy(deg_sh.at[pl.ds(sid * RPT, RPT)],
                  out_hbm.at[cid, pl.ds(sid * RPT, RPT)])


@functools.partial(
    pl.kernel,
    out_type=jax.ShapeDtypeStruct((NC, NP, DW), _f32),
    mesh=_mesh,
    scratch_types=[
        pltpu.VMEM((CPP, CHUNK), jnp.int32),
        pltpu.VMEM((CPP, CHUNK), jnp.int32),
        pltpu.VMEM((CHUNK, DW), _f32),
        pltpu.VMEM((CHUNK, DW), _f32),
        pltpu.VMEM_SHARED((NP, DW), _f32),
        pltpu.SemaphoreType.DMA,
        pltpu.SemaphoreType.DMA,
    ],
)
def _agg_kernel(h_hbm, src_hbm, dst_hbm, out_hbm,
                sidx, didx, rows0, rows1, acc_sh, gsem0, gsem1):
  cid = lax.axis_index("c")
  sid = lax.axis_index("s")

  def zbody(i, _):
    r = i // 8
    c = (i % 8) * 16
    rows0[r, pl.ds(c, 16)] = jnp.zeros((16,), _f32)
    return None
  lax.fori_loop(0, CHUNK * 8, zbody, None)
  for q in range(RPT // CHUNK):
    pltpu.async_copy(rows0, acc_sh.at[pl.ds(sid * RPT + q * CHUNK, CHUNK)],
                     gsem0)
  for q in range(RPT // CHUNK):
    pltpu.make_async_copy(
        rows0, acc_sh.at[pl.ds(sid * RPT + q * CHUNK, CHUNK)], gsem0).wait()
  plsc.subcore_barrier()

  for ph in range(PH):
    if ph:
      pltpu.sync_copy(src_hbm.at[cid, sid, pl.ds(ph * CPP, CPP)], sidx)
      pltpu.sync_copy(dst_hbm.at[cid, sid, pl.ds(ph * CPP, CPP)], didx)
    pltpu.async_copy(h_hbm.at[sidx.at[0]], rows0, gsem0)
    pltpu.async_copy(h_hbm.at[sidx.at[1]], rows1, gsem1)

    def body(g, _):
      for b, rows, sem in ((0, rows0, gsem0), (1, rows1, gsem1)):
        j = 2 * g + b
        pltpu.make_async_copy(h_hbm.at[sidx.at[j]], rows, sem).wait()
        pltpu.sync_copy(rows, acc_sh.at[didx.at[j]], add=True)

        @pl.when(j + 2 < CPP)
        def _():
          pltpu.async_copy(h_hbm.at[sidx.at[j + 2]], rows, sem)
      return None
    lax.fori_loop(0, CPP // 2, body, None)

  plsc.subcore_barrier()
  pltpu.sync_copy(acc_sh.at[pl.ds(sid * RPT, RPT)],
                  out_hbm.at[cid, pl.ds(sid * RPT, RPT)])


def _mm_body(x_ref, w_ref, h_ref):
  h = jnp.dot(x_ref[...], w_ref[...], preferred_element_type=_f32)
  h_ref[...] = jnp.concatenate(
      [h, jnp.zeros((N, DW - D_H), _f32)], axis=1)


def _scale_body(deg_ref, h_ref, dis_ref, hs_ref):
  deg = deg_ref[0] + deg_ref[1]
  dis = lax.rsqrt(deg[:N] + 1.0).reshape(N, 1)
  dis_ref[...] = dis
  hs_ref[...] = h_ref[...] * dis


def _relu_scale_body(aggp_ref, hs_ref, dis_ref, b1_ref, h1s_ref):
  a = aggp_ref[0][:N] + aggp_ref[1][:N]
  b1p = jnp.concatenate(
      [b1_ref[...], jnp.zeros((1, DW - D_H), _f32)], axis=1)
  dis = dis_ref[...]
  h1 = jnp.maximum(dis * (a + hs_ref[...]) + b1p, 0.0)
  h1s_ref[...] = h1 * dis


def _final_body(aggp_ref, h1s_ref, dis_ref, w_ref, b_ref, mu_ref, ls_ref):
  a = aggp_ref[0][:N, :D_H] + aggp_ref[1][:N, :D_H]
  a2 = dis_ref[...] * (a + h1s_ref[:, :D_H])
  out = jnp.dot(a2, w_ref[...], preferred_element_type=_f32) + b_ref[...]
  k = out.shape[1] // 2
  mu_ref[...] = out[:, :k]
  ls_ref[...] = out[:, k:]


def kernel(X, edge_index, W1, b1, W_mu, b_mu, W_ls, b_ls):
  flat = jax.lax.optimization_barrier(edge_index.reshape(2 * E))
  src = flat[:E]
  dst = flat[E:]
  pad = EPAD - E
  ar = jnp.arange(pad, dtype=edge_index.dtype)
  src_f = jnp.concatenate([src, ar % N])
  dst_f = jnp.concatenate([dst, N + ar % PAD_ROWS])
  src_p = src_f.reshape(NC, NS, CPT, CHUNK)
  dst_p = dst_f.reshape(NC, NS, CPT, CHUNK)

  h128 = pl.pallas_call(
      _mm_body,
      out_shape=jax.ShapeDtypeStruct((N, DW), _f32),
  )(X, W1)
  deg_p = _deg_kernel(dst_p)

  dis, hs = pl.pallas_call(
      _scale_body,
      out_shape=(jax.ShapeDtypeStruct((N, 1), _f32),
                 jax.ShapeDtypeStruct((N, DW), _f32)),
  )(deg_p, h128)

  agg1 = _agg_kernel(hs, src_p, dst_p)

  h1s = pl.pallas_call(
      _relu_scale_body,
      out_shape=jax.ShapeDtypeStruct((N, DW), _f32),
  )(agg1, hs, dis, b1.reshape(1, D_H))

  agg2 = _agg_kernel(h1s, src_p, dst_p)

  k = b_mu.shape[0]
  w_cat = jnp.concatenate([W_mu, W_ls], axis=1)
  b_cat = jnp.concatenate([b_mu, b_ls]).reshape(1, 2 * k)
  mu, ls = pl.pallas_call(
      _final_body,
      out_shape=(jax.ShapeDtypeStruct((N, k), _f32),
                 jax.ShapeDtypeStruct((N, k), _f32)),
  )(agg2, h1s, dis, w_cat, b_cat)

  return (mu, ls)
